# baseline (speedup 1.0000x reference)
import functools

import jax
import jax.numpy as jnp
from jax import lax
from jax.experimental import pallas as pl
from jax.experimental.pallas import tpu as pltpu
from jax.experimental.pallas import tpu_sc as plsc

_A = 32
_B = 8192
_D = 128
_NC = 2
_NS = 16
_NW = _NC * _NS
_G = 8
_C = _D // 16
_EPS = 1e-12

_BSC = 4096
_RPW = _BSC // _NW
_NGRP = _RPW // _G
_RTC = 256


def _vrsqrt(s):
    s = jnp.maximum(s, jnp.float32(1e-24))
    i = lax.bitcast_convert_type(s, jnp.int32)
    i = jnp.int32(0x5F3759DF) - lax.shift_right_arithmetic(i, 1)
    y = lax.bitcast_convert_type(i, jnp.float32)
    for _ in range(3):
        y = y * (jnp.float32(1.5) - jnp.float32(0.5) * s * y * y)
    return y


def _sc_body(x_hbm, a_hbm, o_hbm, xv, av0, av1, ov, sem):
    wid = lax.axis_index("s") * _NC + lax.axis_index("c")
    row0 = wid * _RPW
    pltpu.sync_copy(x_hbm.at[pl.ds(row0, _RPW), :], xv)

    def issue(g, av):
        pltpu.async_copy(a_hbm.at[:, pl.ds(row0 + g * _G, _G), :], av, sem)

    def drain(av):
        pltpu.make_async_copy(a_hbm.at[:, pl.ds(0, _G), :], av, sem).wait()

    iota16 = lax.iota(jnp.int32, 16)
    zero16 = jnp.zeros((16,), jnp.float32)

    def compute(g, av):
        def rbody(r, carry):
            xrow = g * _G + r
            xcs = [xv[xrow, pl.ds(16 * c, 16)] for c in range(_C)]
            axc = xcs[0] * xcs[0]
            for c in range(1, _C):
                axc = axc + xcs[c] * xcs[c]
            xsq = jnp.sum(axc)
            rx = _vrsqrt(xsq)

            dv0 = dv1 = nv0 = nv1 = zero16
            for t in range(_A):
                a0 = av[t, r, pl.ds(0, 16)]
                accd = a0 * xcs[0]
                accn = a0 * a0
                for c in range(1, _C):
                    ac = av[t, r, pl.ds(16 * c, 16)]
                    accd = accd + ac * xcs[c]
                    accn = accn + ac * ac
                dot_t = jnp.sum(accd)
                nsq_t = jnp.sum(accn)
                msk = iota16 == (t % 16)
                if t < 16:
                    dv0 = jnp.where(msk, dot_t, dv0)
                    nv0 = jnp.where(msk, nsq_t, nv0)
                else:
                    dv1 = jnp.where(msk, dot_t, dv1)
                    nv1 = jnp.where(msk, nsq_t, nv1)

            rn0 = _vrsqrt(nv0)
            rn1 = _vrsqrt(nv1)
            s0 = dv0 * rn0 * rx
            s1 = dv1 * rn1 * rx
            m = jnp.maximum(jnp.max(s0), jnp.max(s1))
            e0 = jnp.exp(s0 - m)
            e1 = jnp.exp(s1 - m)
            ssum = jnp.sum(e0) + jnp.sum(e1)
            sinv = (zero16 + jnp.float32(1.0)) / (zero16 + ssum)
            w0 = e0 * sinv * rn0
            w1 = e1 * sinv * rn1

            acc = [None] * _C
            for t in range(_A):
                wv = w0 if t < 16 else w1
                wt = jnp.sum(jnp.where(iota16 == (t % 16), wv, zero16))
                for c in range(_C):
                    ac = av[t, r, pl.ds(16 * c, 16)]
                    if t == 0:
                        acc[c] = wt * ac
                    else:
                        acc[c] = acc[c] + wt * ac
            for c in range(_C):
                ov[r, pl.ds(16 * c, 16)] = acc[c]
            return carry

        lax.fori_loop(0, _G, rbody, 0)
        pltpu.sync_copy(ov, o_hbm.at[pl.ds(row0 + g * _G, _G), :])

    issue(0, av0)

    def pair(p, carry):
        g0 = p * 2
        drain(av0)
        issue(g0 + 1, av1)
        compute(g0, av0)
        drain(av1)

        @pl.when(g0 + 2 < _NGRP)
        def _():
            issue(g0 + 2, av0)

        compute(g0 + 1, av1)
        return carry

    lax.fori_loop(0, _NGRP // 2, pair, 0)


_SC_SCRATCH = [
    pltpu.VMEM((_RPW, _D), jnp.float32),
    pltpu.VMEM((_A, _G, _D), jnp.float32),
    pltpu.VMEM((_A, _G, _D), jnp.float32),
    pltpu.VMEM((_G, _D), jnp.float32),
    pltpu.SemaphoreType.DMA,
]


def _sc_run(x, attribute_feat):
    run = functools.partial(
        pl.kernel,
        out_type=jax.ShapeDtypeStruct((_BSC, _D), jnp.float32),
        mesh=plsc.VectorSubcoreMesh(core_axis_name="c", subcore_axis_name="s"),
        compiler_params=pltpu.CompilerParams(needs_layout_passes=False),
        scratch_types=_SC_SCRATCH,
    )(_sc_body)
    return run(x, attribute_feat)


def _tc_body(x_ref, a_ref, o_ref, dots_ref, nsq_ref, coef_ref):
    xb = x_ref[...]
    a = a_ref[...]
    xnsq = jnp.sum(xb * xb, axis=1, keepdims=True)
    xinv = 1.0 / jnp.maximum(jnp.sqrt(xnsq), _EPS)
    xn = xb * xinv
    dots_ref[...] = jnp.sum(a * xn[None, :, :], axis=2)
    nsq_ref[...] = jnp.sum(a * a, axis=2)
    dots = dots_ref[...]
    nsq = nsq_ref[...]
    nainv = 1.0 / jnp.maximum(jnp.sqrt(nsq), _EPS)
    scores = dots * nainv
    m = jnp.max(scores, axis=0, keepdims=True)
    e = jnp.exp(scores - m)
    sinv = 1.0 / jnp.sum(e, axis=0, keepdims=True)
    coef_ref[...] = e * sinv * nainv
    coef = coef_ref[...]
    o_ref[...] = jnp.sum(a * coef[:, :, None], axis=0)


_BLK0 = _BSC // _RTC


def _tc_run(x, attribute_feat):
    return pl.pallas_call(
        _tc_body,
        grid=((_B - _BSC) // _RTC,),
        in_specs=[
            pl.BlockSpec((_RTC, _D), lambda i: (i + _BLK0, 0)),
            pl.BlockSpec((_A, _RTC, _D), lambda i: (0, i + _BLK0, 0)),
        ],
        out_specs=pl.BlockSpec((_RTC, _D), lambda i: (i, 0)),
        out_shape=jax.ShapeDtypeStruct((_B - _BSC, _D), jnp.float32),
        scratch_shapes=[
            pltpu.VMEM((_A, _RTC), jnp.float32),
            pltpu.VMEM((_A, _RTC), jnp.float32),
            pltpu.VMEM((_A, _RTC), jnp.float32),
        ],
    )(x, attribute_feat)


def kernel(x, attribute_feat):
    out_sc = _sc_run(x, attribute_feat)
    out_tc = _tc_run(x, attribute_feat)
    return jnp.concatenate([out_sc, out_tc], axis=0)


# speedup vs baseline: 1.0676x; 1.0676x over previous
import functools

import jax
import jax.numpy as jnp
from jax import lax
from jax.experimental import pallas as pl
from jax.experimental.pallas import tpu as pltpu
from jax.experimental.pallas import tpu_sc as plsc

_A = 32
_B = 8192
_D = 128
_NC = 2
_NS = 16
_NW = _NC * _NS
_G = 8
_C = _D // 16
_EPS = 1e-12

_BSC = 3584
_RPW = _BSC // _NW
_NGRP = _RPW // _G
_RTC = 256


def _vrsqrt(s):
    s = jnp.maximum(s, jnp.float32(1e-24))
    i = lax.bitcast_convert_type(s, jnp.int32)
    i = jnp.int32(0x5F3759DF) - lax.shift_right_arithmetic(i, 1)
    y = lax.bitcast_convert_type(i, jnp.float32)
    for _ in range(3):
        y = y * (jnp.float32(1.5) - jnp.float32(0.5) * s * y * y)
    return y


def _sc_body(x_hbm, a_hbm, o_hbm, xv, av0, av1, ov, sem):
    wid = lax.axis_index("s") * _NC + lax.axis_index("c")
    row0 = wid * _RPW
    pltpu.sync_copy(x_hbm.at[pl.ds(row0, _RPW), :], xv)

    def issue(g, av):
        pltpu.async_copy(a_hbm.at[:, pl.ds(row0 + g * _G, _G), :], av, sem)

    def drain(av):
        pltpu.make_async_copy(a_hbm.at[:, pl.ds(0, _G), :], av, sem).wait()

    iota16 = lax.iota(jnp.int32, 16)
    zero16 = jnp.zeros((16,), jnp.float32)

    def compute(g, av):
        def rbody(r, carry):
            xrow = g * _G + r
            xcs = [xv[xrow, pl.ds(16 * c, 16)] for c in range(_C)]
            axc = xcs[0] * xcs[0]
            for c in range(1, _C):
                axc = axc + xcs[c] * xcs[c]
            xsq = jnp.sum(axc)
            rx = _vrsqrt(xsq)

            dv0 = dv1 = nv0 = nv1 = zero16
            for t in range(_A):
                a0 = av[t, r, pl.ds(0, 16)]
                accd = a0 * xcs[0]
                accn = a0 * a0
                for c in range(1, _C):
                    ac = av[t, r, pl.ds(16 * c, 16)]
                    accd = accd + ac * xcs[c]
                    accn = accn + ac * ac
                dot_t = jnp.sum(accd)
                nsq_t = jnp.sum(accn)
                msk = iota16 == (t % 16)
                if t < 16:
                    dv0 = jnp.where(msk, dot_t, dv0)
                    nv0 = jnp.where(msk, nsq_t, nv0)
                else:
                    dv1 = jnp.where(msk, dot_t, dv1)
                    nv1 = jnp.where(msk, nsq_t, nv1)

            rn0 = _vrsqrt(nv0)
            rn1 = _vrsqrt(nv1)
            s0 = dv0 * rn0 * rx
            s1 = dv1 * rn1 * rx
            m = jnp.maximum(jnp.max(s0), jnp.max(s1))
            e0 = jnp.exp(s0 - m)
            e1 = jnp.exp(s1 - m)
            ssum = jnp.sum(e0) + jnp.sum(e1)
            sinv = (zero16 + jnp.float32(1.0)) / (zero16 + ssum)
            w0 = e0 * sinv * rn0
            w1 = e1 * sinv * rn1

            acc = [None] * _C
            for t in range(_A):
                wv = w0 if t < 16 else w1
                wt = jnp.sum(jnp.where(iota16 == (t % 16), wv, zero16))
                for c in range(_C):
                    ac = av[t, r, pl.ds(16 * c, 16)]
                    if t == 0:
                        acc[c] = wt * ac
                    else:
                        acc[c] = acc[c] + wt * ac
            for c in range(_C):
                ov[r, pl.ds(16 * c, 16)] = acc[c]
            return carry

        lax.fori_loop(0, _G, rbody, 0)
        pltpu.sync_copy(ov, o_hbm.at[pl.ds(row0 + g * _G, _G), :])

    issue(0, av0)

    def pair(p, carry):
        g0 = p * 2
        drain(av0)
        issue(g0 + 1, av1)
        compute(g0, av0)
        drain(av1)

        @pl.when(g0 + 2 < _NGRP)
        def _():
            issue(g0 + 2, av0)

        compute(g0 + 1, av1)
        return carry

    lax.fori_loop(0, _NGRP // 2, pair, 0)


_SC_SCRATCH = [
    pltpu.VMEM((_RPW, _D), jnp.float32),
    pltpu.VMEM((_A, _G, _D), jnp.float32),
    pltpu.VMEM((_A, _G, _D), jnp.float32),
    pltpu.VMEM((_G, _D), jnp.float32),
    pltpu.SemaphoreType.DMA,
]


def _sc_run(x, attribute_feat):
    run = functools.partial(
        pl.kernel,
        out_type=jax.ShapeDtypeStruct((_BSC, _D), jnp.float32),
        mesh=plsc.VectorSubcoreMesh(core_axis_name="c", subcore_axis_name="s"),
        compiler_params=pltpu.CompilerParams(needs_layout_passes=False),
        scratch_types=_SC_SCRATCH,
    )(_sc_body)
    return run(x, attribute_feat)


def _tc_body(x_ref, a_ref, o_ref, dots_ref, nsq_ref, coef_ref):
    xb = x_ref[...]
    a = a_ref[...]
    xnsq = jnp.sum(xb * xb, axis=1, keepdims=True)
    xinv = 1.0 / jnp.maximum(jnp.sqrt(xnsq), _EPS)
    xn = xb * xinv
    dots_ref[...] = jnp.sum(a * xn[None, :, :], axis=2)
    nsq_ref[...] = jnp.sum(a * a, axis=2)
    dots = dots_ref[...]
    nsq = nsq_ref[...]
    nainv = 1.0 / jnp.maximum(jnp.sqrt(nsq), _EPS)
    scores = dots * nainv
    m = jnp.max(scores, axis=0, keepdims=True)
    e = jnp.exp(scores - m)
    sinv = 1.0 / jnp.sum(e, axis=0, keepdims=True)
    coef_ref[...] = e * sinv * nainv
    coef = coef_ref[...]
    o_ref[...] = jnp.sum(a * coef[:, :, None], axis=0)


_BLK0 = _BSC // _RTC


def _tc_run(x, attribute_feat):
    return pl.pallas_call(
        _tc_body,
        grid=((_B - _BSC) // _RTC,),
        in_specs=[
            pl.BlockSpec((_RTC, _D), lambda i: (i + _BLK0, 0)),
            pl.BlockSpec((_A, _RTC, _D), lambda i: (0, i + _BLK0, 0)),
        ],
        out_specs=pl.BlockSpec((_RTC, _D), lambda i: (i, 0)),
        out_shape=jax.ShapeDtypeStruct((_B - _BSC, _D), jnp.float32),
        scratch_shapes=[
            pltpu.VMEM((_A, _RTC), jnp.float32),
            pltpu.VMEM((_A, _RTC), jnp.float32),
            pltpu.VMEM((_A, _RTC), jnp.float32),
        ],
    )(x, attribute_feat)


def kernel(x, attribute_feat):
    out_sc = _sc_run(x, attribute_feat)
    out_tc = _tc_run(x, attribute_feat)
    return jnp.concatenate([out_sc, out_tc], axis=0)
